# baseline (speedup 1.0000x reference)
import functools

import jax
import jax.numpy as jnp
from jax import lax
from jax.experimental import pallas as pl
from jax.experimental.pallas import tpu as pltpu
from jax.experimental.pallas import tpu_sc as plsc

_B = 16384
_F = 32
_NW = 32
_BPW = _B // _NW
_G = 16


def _gmf_body(user_hbm, item_hbm, euT_hbm, eiT_hbm, w_hbm, b_hbm, out_hbm,
              uidx_v, iidx_v, blk_v, cols_v, w_v, b_v, out_v, sem):
    wid = lax.axis_index("s") * 2 + lax.axis_index("c")
    base = wid * _BPW

    pltpu.sync_copy(user_hbm.at[pl.ds(base, _BPW)], uidx_v)
    pltpu.sync_copy(item_hbm.at[pl.ds(base, _BPW)], iidx_v)
    pltpu.sync_copy(w_hbm, w_v)
    pltpu.sync_copy(b_hbm, b_v)

    w0 = w_v[0, :]
    w1 = w_v[1, :]
    ws = [w0[f] for f in range(16)] + [w1[f] for f in range(16)]
    bias = b_v[:]
    e_iota = lax.iota(jnp.int32, _G)
    zero = jnp.zeros((_G,), jnp.float32)

    def fetch(idx_vec):
        tc = lax.shift_right_logical(idx_vec, 7)
        copies = []
        for e in range(_G):
            c0 = pl.multiple_of(tc[e] * 128, 128)
            copies.append(pltpu.async_copy(
                euT_hbm.at[:, pl.ds(c0, 128)], blk_v.at[e], sem))
        for c in copies:
            c.wait()

    def fetch_i(idx_vec):
        tc = lax.shift_right_logical(idx_vec, 7)
        copies = []
        for e in range(_G):
            c0 = pl.multiple_of(tc[e] * 128, 128)
            copies.append(pltpu.async_copy(
                eiT_hbm.at[:, pl.ds(c0, 128)], blk_v.at[e], sem))
        for c in copies:
            c.wait()

    def body(g, carry):
        off = pl.multiple_of(g * _G, _G)
        uvec = uidx_v[pl.ds(off, _G)]
        ivec = iidx_v[pl.ds(off, _G)]
        lanes_u = uvec & jnp.int32(127)
        lanes_i = ivec & jnp.int32(127)

        fetch(uvec)
        for f in range(_F):
            f_splat = jnp.full((_G,), f, jnp.int32)
            cols_v[f] = plsc.load_gather(blk_v, [e_iota, f_splat, lanes_u])

        fetch_i(ivec)
        acc = zero
        for f in range(_F):
            f_splat = jnp.full((_G,), f, jnp.int32)
            gi = plsc.load_gather(blk_v, [e_iota, f_splat, lanes_i])
            acc = acc + gi * cols_v[f] * ws[f]
        out_v[pl.ds(off, _G)] = acc + bias
        return carry

    lax.fori_loop(0, _BPW // _G, body, 0)

    pltpu.sync_copy(out_v, out_hbm.at[pl.ds(base, _BPW)])


_gmf = functools.partial(
    pl.kernel,
    out_type=jax.ShapeDtypeStruct((_B,), jnp.float32),
    mesh=plsc.VectorSubcoreMesh(core_axis_name="c", subcore_axis_name="s"),
    compiler_params=pltpu.CompilerParams(disable_bounds_checks=True,
                                         needs_layout_passes=False),
    scratch_types=[
        pltpu.VMEM((_BPW,), jnp.int32),
        pltpu.VMEM((_BPW,), jnp.int32),
        pltpu.VMEM((_G, _F, 128), jnp.float32),
        pltpu.VMEM((_F, _G), jnp.float32),
        pltpu.VMEM((2, 16), jnp.float32),
        pltpu.VMEM((16,), jnp.float32),
        pltpu.VMEM((_BPW,), jnp.float32),
        pltpu.SemaphoreType.DMA,
    ],
)(_gmf_body)


def kernel(user, item, embed_user, embed_item, predict_w, predict_b):
    w2 = predict_w.reshape(2, 16)
    bvec = jnp.broadcast_to(predict_b, (16,))
    return _gmf(user, item, embed_user.T, embed_item.T, w2, bvec)
